# baseline (speedup 1.0000x reference)
import jax
import jax.numpy as jnp
from jax.experimental import pallas as pl
from jax.experimental.pallas import tpu as pltpu

TM = 1024
NSPLIT = 4


def _router_block(*refs):
    x_refs = refs[:NSPLIT]
    wt_refs = refs[NSPLIT:2 * NSPLIT]
    b_ref = refs[2 * NSPLIT]
    out_ref = refs[2 * NSPLIT + 1]
    logits = b_ref[...]
    for xr, wr in zip(x_refs, wt_refs):
        logits = logits + jnp.dot(
            xr[...], wr[...], preferred_element_type=jnp.float32)
    m = jnp.max(logits, axis=-1, keepdims=True)
    e = jnp.exp(logits - m)
    out_ref[...] = e / jnp.sum(e, axis=-1, keepdims=True)


def kernel(x, W, b):
    tokens, d_model = x.shape
    num_experts = W.shape[0]
    kc = d_model // NSPLIT
    wt = W.T
    b2 = b.reshape(1, num_experts)
    grid = (tokens // TM,)
    x_specs = [
        pl.BlockSpec((TM, kc), lambda i, j=j: (i, j)) for j in range(NSPLIT)
    ]
    wt_specs = [
        pl.BlockSpec((kc, num_experts), lambda i, j=j: (j, 0))
        for j in range(NSPLIT)
    ]
    return pl.pallas_call(
        _router_block,
        grid=grid,
        in_specs=x_specs + wt_specs + [
            pl.BlockSpec((1, num_experts), lambda i: (0, 0)),
        ],
        out_specs=pl.BlockSpec((TM, num_experts), lambda i: (i, 0)),
        out_shape=jax.ShapeDtypeStruct((tokens, num_experts), jnp.float32),
        compiler_params=pltpu.CompilerParams(
            dimension_semantics=("arbitrary",),
        ),
    )(*([x] * NSPLIT), *([wt] * NSPLIT), b2)
